# baseline (speedup 1.0000x reference)
import functools

import jax
import jax.numpy as jnp
from jax import lax
from jax.experimental import pallas as pl
from jax.experimental.pallas import tpu as pltpu
from jax.experimental.pallas import tpu_sc as plsc

BN = 2048
NCHUNK = 4
SC_CHUNK = 128


def _dist_body(x_ref, e_ref, dist_ref, ind_ref):
    xb = x_ref[...]
    e = e_ref[...]
    c = e.shape[0]
    d = jax.lax.dot_general(xb, e, (((1,), (1,)), ((), ())),
                            preferred_element_type=jnp.float32)
    dist_ref[...] = d
    m = jnp.max(d, axis=-1, keepdims=True)
    iota = jax.lax.broadcasted_iota(jnp.int32, d.shape, 1).astype(jnp.float32)
    w = jnp.where(d == m, iota, float(c))
    idx = jnp.min(w, axis=-1, keepdims=True)
    ind_ref[0, 0, :] = jnp.transpose(idx.astype(jnp.int32), (1, 0))[0]


def _dist_body_alias(x_ref, e_ref, dprev_ref, dist_ref, ind_ref):
    del dprev_ref
    _dist_body(x_ref, e_ref, dist_ref, ind_ref)


def _alloc_body(o_ref):
    del o_ref


def _make_sc_gather(n_sub, row_off, dim):
    info = plsc.get_sparse_core_info()
    nw = info.num_cores * info.num_subcores
    per_w = n_sub // nw
    n_inner = per_w // SC_CHUNK
    mesh = plsc.VectorSubcoreMesh(core_axis_name="c", subcore_axis_name="s")

    @functools.partial(
        pl.kernel,
        mesh=mesh,
        out_type=(),
        scratch_types=[
            pltpu.VMEM((SC_CHUNK,), jnp.int32),
            pltpu.VMEM((SC_CHUNK, dim), jnp.float32),
            pltpu.SemaphoreType.DMA,
        ],
    )
    def sc_gather(idx_hbm, table_hbm, q_hbm, idx_v, rows_v, sem):
        wid = lax.axis_index("s") * info.num_cores + lax.axis_index("c")
        for j in range(n_inner):
            base = wid * per_w + j * SC_CHUNK
            pltpu.sync_copy(idx_hbm.at[pl.ds(base, SC_CHUNK)], idx_v)
            pltpu.async_copy(table_hbm.at[idx_v], rows_v, sem).wait()
            pltpu.sync_copy(rows_v, q_hbm.at[pl.ds(row_off + base, SC_CHUNK)])

    return sc_gather


def kernel(x, embed):
    x = x.astype(jnp.float32)
    b, n, d = x.shape
    h, c, _ = embed.shape
    N = b * n
    xf = x.reshape(N, d)
    ef = embed.reshape(c, d)
    gtot = N // BN
    cpb = gtot // NCHUNK
    rows_per_chunk = cpb * BN

    qbuf = pl.pallas_call(
        _alloc_body,
        out_specs=pl.BlockSpec(memory_space=pl.ANY),
        out_shape=jax.ShapeDtypeStruct((N, d), jnp.float32),
    )()
    qref = jax.new_ref(qbuf)

    def tc_chunk(k, dist_prev):
        in_specs = [
            pl.BlockSpec((BN, d), lambda i, k=k: (i + k * cpb, 0)),
            pl.BlockSpec((c, d), lambda i: (0, 0)),
        ]
        args = [xf, ef]
        io_alias = {}
        body = _dist_body
        if dist_prev is not None:
            in_specs.append(pl.BlockSpec(memory_space=pl.ANY))
            args.append(dist_prev)
            io_alias = {2: 0}
            body = _dist_body_alias
        return pl.pallas_call(
            body,
            grid=(cpb,),
            in_specs=in_specs,
            out_specs=[
                pl.BlockSpec((BN, c), lambda i, k=k: (i + k * cpb, 0)),
                pl.BlockSpec((1, 1, BN), lambda i: (i, 0, 0)),
            ],
            out_shape=[
                jax.ShapeDtypeStruct((N, c), jnp.float32),
                jax.ShapeDtypeStruct((cpb, 1, BN), jnp.int32),
            ],
            input_output_aliases=io_alias,
        )(*args)

    dist = None
    inds = []
    for k in range(NCHUNK):
        dist, ind_k = tc_chunk(k, dist)
        _make_sc_gather(rows_per_chunk, k * rows_per_chunk, d)(
            ind_k.reshape(rows_per_chunk), ef, qref)
        inds.append(ind_k)

    quantize = qref[...].reshape(b, n, d)
    embed_ind = jnp.concatenate(inds, axis=0).reshape(b, n)
    dist_out = dist.reshape(h, b, n, c)
    return quantize, embed_ind, dist_out


# speedup vs baseline: 1.4686x; 1.4686x over previous
import jax
import jax.numpy as jnp
from jax.experimental import pallas as pl
from jax.experimental.pallas import tpu as pltpu

BN = 2048
IW = 128


def _body(x_ref, e_ref, dist_ref, ind_ref, q_ref):
    xb = x_ref[...]
    e = e_ref[...]
    c = e.shape[0]
    d = jax.lax.dot_general(xb, e, (((1,), (1,)), ((), ())),
                            preferred_element_type=jnp.float32)
    dist_ref[...] = d
    m = jnp.max(d, axis=-1, keepdims=True)
    iota = jax.lax.broadcasted_iota(jnp.int32, d.shape, 1).astype(jnp.float32)
    w = jnp.where(d == m, iota, float(c))
    idx = jnp.min(w, axis=-1, keepdims=True)
    ind_ref[0, 0, :] = jnp.transpose(idx.astype(jnp.int32), (1, 0))[0]
    oh = (iota == idx).astype(jnp.float32)
    q_ref[...] = jax.lax.dot_general(oh, e, (((1,), (0,)), ((), ())),
                                     preferred_element_type=jnp.float32)


def kernel(x, embed):
    x = x.astype(jnp.float32)
    b, n, d = x.shape
    h, c, _ = embed.shape
    N = b * n
    xf = x.reshape(N, d)
    ef = embed.reshape(c, d)
    dist, ind_wide, quant = pl.pallas_call(
        _body,
        grid=(N // BN,),
        in_specs=[
            pl.BlockSpec((BN, d), lambda i: (i, 0)),
            pl.BlockSpec((c, d), lambda i: (0, 0)),
        ],
        out_specs=[
            pl.BlockSpec((BN, c), lambda i: (i, 0)),
            pl.BlockSpec((1, 1, BN), lambda i: (i, 0, 0)),
            pl.BlockSpec((BN, d), lambda i: (i, 0)),
        ],
        out_shape=[
            jax.ShapeDtypeStruct((N, c), jnp.float32),
            jax.ShapeDtypeStruct((N // BN, 1, BN), jnp.int32),
            jax.ShapeDtypeStruct((N, d), jnp.float32),
        ],
    )(xf, ef)
    quantize = quant.reshape(b, n, d)
    embed_ind = ind_wide.reshape(b, n)
    dist_out = dist.reshape(h, b, n, c)
    return quantize, embed_ind, dist_out


# speedup vs baseline: 1.7099x; 1.1643x over previous
import jax
import jax.numpy as jnp
from jax.experimental import pallas as pl
from jax.experimental.pallas import tpu as pltpu

BN = 2048


def _body(x_ref, e_ref, dist_ref, ind_ref, q_ref, idx_s):
    i = pl.program_id(0)
    g = pl.num_programs(0) - 1
    e = e_ref[...]
    c = e.shape[0]

    @pl.when(i > 0)
    def _quant_prev():
        idxp = idx_s[...]
        iota = jax.lax.broadcasted_iota(
            jnp.int32, (idxp.shape[0], c), 1).astype(jnp.float32)
        oh = (iota == idxp).astype(jnp.float32)
        q_ref[...] = jax.lax.dot_general(oh, e, (((1,), (0,)), ((), ())),
                                         preferred_element_type=jnp.float32)

    @pl.when(i < g)
    def _dist_cur():
        xb = x_ref[...]
        d = jax.lax.dot_general(xb, e, (((1,), (1,)), ((), ())),
                                preferred_element_type=jnp.float32)
        dist_ref[...] = d
        m = jnp.max(d, axis=-1, keepdims=True)
        iota = jax.lax.broadcasted_iota(
            jnp.int32, d.shape, 1).astype(jnp.float32)
        w = jnp.where(d == m, iota, float(c))
        idx = jnp.min(w, axis=-1, keepdims=True)
        ind_ref[0, 0, :] = jnp.transpose(idx.astype(jnp.int32), (1, 0))[0]
        idx_s[...] = idx


def kernel(x, embed):
    x = x.astype(jnp.float32)
    b, n, d = x.shape
    h, c, _ = embed.shape
    N = b * n
    xf = x.reshape(N, d)
    ef = embed.reshape(c, d)
    g = N // BN
    last = g - 1
    dist, ind3, quant = pl.pallas_call(
        _body,
        grid=(g + 1,),
        in_specs=[
            pl.BlockSpec((BN, d), lambda i: (jnp.minimum(i, last), 0)),
            pl.BlockSpec((c, d), lambda i: (0, 0)),
        ],
        out_specs=[
            pl.BlockSpec((BN, c), lambda i: (jnp.minimum(i, last), 0)),
            pl.BlockSpec((1, 1, BN), lambda i: (jnp.minimum(i, last), 0, 0)),
            pl.BlockSpec((BN, d), lambda i: (jnp.maximum(i, 1) - 1, 0)),
        ],
        out_shape=[
            jax.ShapeDtypeStruct((N, c), jnp.float32),
            jax.ShapeDtypeStruct((g, 1, BN), jnp.int32),
            jax.ShapeDtypeStruct((N, d), jnp.float32),
        ],
        scratch_shapes=[pltpu.VMEM((BN, 1), jnp.float32)],
    )(xf, ef)
    quantize = quant.reshape(b, n, d)
    embed_ind = ind3.reshape(b, n)
    dist_out = dist.reshape(h, b, n, c)
    return quantize, embed_ind, dist_out
